# baseline (speedup 1.0000x reference)
import functools

import jax
import jax.numpy as jnp
from jax import lax
from jax.experimental import pallas as pl
from jax.experimental.pallas import tpu as pltpu
from jax.experimental.pallas import tpu_sc as plsc

_MAX_BOX = 100
_IOU_THRES = 0.5
_SCORE_THRES = 0.05
_NEG = -1e10
_L = 16
_KP = 112


def _rot(buf, v, sh):
    buf[pl.ds(0, _L)] = v
    buf[pl.ds(_L, _L)] = v
    return buf[pl.ds(sh, _L)]


def _argmax_bcast(key, idx, payloads, rotf, roti):
    for sh in (8, 4, 2, 1):
        pk = _rot(rotf, key, sh)
        pi = _rot(roti, idx, sh)
        take = (pk > key) | ((pk == key) & (pi < idx))
        key = jnp.where(take, pk, key)
        idx = jnp.where(take, pi, idx)
        payloads = [jnp.where(take, _rot(rotf, p, sh), p)
                    for p in payloads]
    return key, idx, payloads


def _nms_sc_body(C, RS, B, BP, H, W, NW,
                 sc_hbm, ro_hbm, dl_hbm,
                 sels_hbm, sy1_hbm, sx1_hbm, sy2_hbm, sx2_hbm,
                 sv, ry1, rx1, ry2, rx2, d0, d1, d2, d3,
                 y1v, x1v, y2v, x2v, a2v,
                 sel_s, sel_y1, sel_x1, sel_y2, sel_x2, rotf, roti):
    NB = BP // _L
    f32 = jnp.float32
    iota = lax.iota(jnp.int32, _L)
    wid = lax.axis_index("s") * 2 + lax.axis_index("c")

    def do_class(r):
        n = r // C
        pltpu.sync_copy(sc_hbm.at[pl.ds(r * BP, BP)], sv)
        pltpu.sync_copy(ro_hbm.at[pl.ds((n * 4 + 0) * BP, BP)], ry1)
        pltpu.sync_copy(ro_hbm.at[pl.ds((n * 4 + 1) * BP, BP)], rx1)
        pltpu.sync_copy(ro_hbm.at[pl.ds((n * 4 + 2) * BP, BP)], ry2)
        pltpu.sync_copy(ro_hbm.at[pl.ds((n * 4 + 3) * BP, BP)], rx2)
        pltpu.sync_copy(dl_hbm.at[pl.ds((r * 4 + 0) * BP, BP)], d0)
        pltpu.sync_copy(dl_hbm.at[pl.ds((r * 4 + 1) * BP, BP)], d1)
        pltpu.sync_copy(dl_hbm.at[pl.ds((r * 4 + 2) * BP, BP)], d2)
        pltpu.sync_copy(dl_hbm.at[pl.ds((r * 4 + 3) * BP, BP)], d3)

        def dec(i, carry):
            mx, mi, py1, px1, py2, px2 = carry
            sl = pl.ds(i * _L, _L)
            a = ry1[sl]
            b = rx1[sl]
            c2 = ry2[sl]
            e = rx2[sl]
            w0 = e - b + 1.0
            h0 = c2 - a + 1.0
            x0 = b + w0 / 2.0
            y0 = a + h0 / 2.0
            cx = (d0[sl] / 10.0) * w0 + x0
            cy = (d1[sl] / 10.0) * h0 + y0
            wwv = jnp.exp(d2[sl] / 5.0) * w0
            hhv = jnp.exp(d3[sl] / 5.0) * h0
            xx1 = jnp.clip(cx - 0.5 * wwv, 0.0, W - 1.0)
            yy1 = jnp.clip(cy - 0.5 * hhv, 0.0, H - 1.0)
            xx2 = jnp.clip(cx + 0.5 * wwv, 0.0, W - 1.0)
            yy2 = jnp.clip(cy + 0.5 * hhv, 0.0, H - 1.0)
            y1v[sl] = yy1
            x1v[sl] = xx1
            y2v[sl] = yy2
            x2v[sl] = xx2
            a2v[sl] = (yy2 - yy1) * (xx2 - xx1)
            s = sv[sl]
            gidx = jnp.broadcast_to(i * _L, (_L,)) + iota
            keep = (s > _SCORE_THRES) & (gidx < B)
            snew = jnp.where(keep, s, _NEG)
            sv[sl] = snew
            take = snew > mx
            return (jnp.where(take, snew, mx), jnp.where(take, gidx, mi),
                    jnp.where(take, yy1, py1), jnp.where(take, xx1, px1),
                    jnp.where(take, yy2, py2), jnp.where(take, xx2, px2))

        zf0 = jnp.zeros((_L,), f32)
        mx, mi, py1, px1, py2, px2 = lax.fori_loop(
            0, NB, dec,
            (jnp.full((_L,), -2e10, f32), jnp.zeros((_L,), jnp.int32),
             zf0, zf0, zf0, zf0))
        Mv0, giv0, (by10, bx10, by20, bx20) = _argmax_bcast(
            mx, mi, [py1, px1, py2, px2], rotf, roti)

        def step(k, carry):
            (Mv, giv, by1, bx1, by2, bx2,
             acc_s, acc_y1, acc_x1, acc_y2, acc_x2) = carry

            validv = Mv > (_NEG * 0.5)
            kmod = k - (k // _L) * _L
            hit = iota == jnp.broadcast_to(kmod, (_L,))
            acc_s2 = jnp.where(hit, jnp.where(validv, Mv, 0.0), acc_s)
            acc_y12 = jnp.where(hit, jnp.where(validv, by1, 0.0), acc_y1)
            acc_x12 = jnp.where(hit, jnp.where(validv, bx1, 0.0), acc_x1)
            acc_y22 = jnp.where(hit, jnp.where(validv, by2, 0.0), acc_y2)
            acc_x22 = jnp.where(hit, jnp.where(validv, bx2, 0.0), acc_x2)
            flush = (kmod == _L - 1) | (k == _MAX_BOX - 1)

            @pl.when(flush)
            def _():
                base = k - kmod
                sel_s[pl.ds(base, _L)] = acc_s2
                sel_y1[pl.ds(base, _L)] = acc_y12
                sel_x1[pl.ds(base, _L)] = acc_x12
                sel_y2[pl.ds(base, _L)] = acc_y22
                sel_x2[pl.ds(base, _L)] = acc_x22

            a1 = (by2 - by1) * (bx2 - bx1)

            def sweep4(i, c):
                slots = []
                for u in range(6):
                    mx, mi, py1, px1, py2, px2 = c[6 * u:6 * u + 6]
                    blk = i * 6 + u
                    sl = pl.ds(blk * _L, _L)
                    y1 = y1v[sl]
                    x1 = x1v[sl]
                    y2 = y2v[sl]
                    x2 = x2v[sl]
                    yA = jnp.maximum(by1, y1)
                    xA = jnp.maximum(bx1, x1)
                    yB = jnp.minimum(by2, y2)
                    xB = jnp.minimum(bx2, x2)
                    inter = (jnp.maximum(yB - yA, 0.0)
                             * jnp.maximum(xB - xA, 0.0))
                    iou = inter / (a1 + a2v[sl] - inter + 1e-8)
                    gidx = jnp.broadcast_to(blk * _L, (_L,)) + iota
                    s = sv[sl]
                    snew = jnp.where((iou > _IOU_THRES) | (gidx == giv),
                                     _NEG, s)
                    sv[sl] = snew
                    take = snew > mx
                    slots.extend(
                        (jnp.where(take, snew, mx),
                         jnp.where(take, gidx, mi),
                         jnp.where(take, y1, py1),
                         jnp.where(take, x1, px1),
                         jnp.where(take, y2, py2),
                         jnp.where(take, x2, px2)))
                return tuple(slots)

            zf = jnp.zeros((_L,), f32)
            init1 = (jnp.full((_L,), -2e10, f32),
                     jnp.zeros((_L,), jnp.int32), zf, zf, zf, zf)
            out4 = lax.fori_loop(0, NB // 6, sweep4, init1 * 6)

            def comb(sa, sb):
                take = ((sb[0] > sa[0])
                        | ((sb[0] == sa[0]) & (sb[1] < sa[1])))
                return tuple(jnp.where(take, b, a)
                             for a, b in zip(sa, sb))

            s01 = comb(out4[0:6], out4[6:12])
            s23 = comb(out4[12:18], out4[18:24])
            s45 = comb(out4[24:30], out4[30:36])
            mx, mi, py1, px1, py2, px2 = comb(comb(s01, s23), s45)
            nMv, ngiv, (nby1, nbx1, nby2, nbx2) = _argmax_bcast(
                mx, mi, [py1, px1, py2, px2], rotf, roti)

            flvf = jnp.broadcast_to(jnp.where(flush, 1.0, 0.0), (_L,))
            keepf = 1.0 - flvf
            return (nMv, ngiv, nby1, nbx1, nby2, nbx2,
                    acc_s2 * keepf - flvf,
                    acc_y12 * keepf,
                    acc_x12 * keepf,
                    acc_y22 * keepf,
                    acc_x22 * keepf)

        lax.fori_loop(0, _MAX_BOX, step,
                      (Mv0, giv0, by10, bx10, by20, bx20,
                       jnp.full((_L,), -1.0, f32), zf0, zf0, zf0, zf0))

        pltpu.sync_copy(sel_s, sels_hbm.at[pl.ds(r * _KP, _KP)])
        pltpu.sync_copy(sel_y1, sy1_hbm.at[pl.ds(r * _KP, _KP)])
        pltpu.sync_copy(sel_x1, sx1_hbm.at[pl.ds(r * _KP, _KP)])
        pltpu.sync_copy(sel_y2, sy2_hbm.at[pl.ds(r * _KP, _KP)])
        pltpu.sync_copy(sel_x2, sx2_hbm.at[pl.ds(r * _KP, _KP)])

    for j in range(-(-RS // NW)):
        r = wid + NW * j

        @pl.when(r < RS)
        def _():
            do_class(r)


def _nms_tc_body(H, W, B,
                 s_in, ry1, rx1, ry2, rx2, t0, t1, t2, t3,
                 oss, osy1, osx1, osy2, osx2,
                 sref, y1s, x1s, y2s, x2s, a2s):
    RT, BPT = s_in.shape
    f32 = jnp.float32

    w0 = rx2[...] - rx1[...] + 1.0
    h0 = ry2[...] - ry1[...] + 1.0
    x0 = rx1[...] + w0 / 2.0
    y0 = ry1[...] + h0 / 2.0
    cx = (t0[...] / 10.0) * w0 + x0
    cy = (t1[...] / 10.0) * h0 + y0
    ww = jnp.exp(t2[...] / 5.0) * w0
    hh = jnp.exp(t3[...] / 5.0) * h0
    xx1 = jnp.clip(cx - 0.5 * ww, 0.0, W - 1.0)
    yy1 = jnp.clip(cy - 0.5 * hh, 0.0, H - 1.0)
    xx2 = jnp.clip(cx + 0.5 * ww, 0.0, W - 1.0)
    yy2 = jnp.clip(cy + 0.5 * hh, 0.0, H - 1.0)
    y1s[...] = yy1
    x1s[...] = xx1
    y2s[...] = yy2
    x2s[...] = xx2
    a2s[...] = (yy2 - yy1) * (xx2 - xx1)

    lane = jax.lax.broadcasted_iota(jnp.int32, (RT, BPT), 1)
    s = s_in[...]
    sref[...] = jnp.where((lane < B) & (s > _SCORE_THRES), s, _NEG)

    lane128 = jax.lax.broadcasted_iota(jnp.int32, (RT, 128), 1)

    def nms_step(k, carry):
        ss, sy1, sx1, sy2, sx2 = carry
        s = sref[...]
        best = jnp.max(s, axis=1, keepdims=True)
        idx = jnp.min(jnp.where(s == best, lane, BPT), axis=1,
                      keepdims=True)
        eq = lane == idx
        eqf = eq.astype(f32)
        y1 = y1s[...]
        x1 = x1s[...]
        y2 = y2s[...]
        x2 = x2s[...]
        by1 = jnp.sum(y1 * eqf, axis=1, keepdims=True)
        bx1 = jnp.sum(x1 * eqf, axis=1, keepdims=True)
        by2 = jnp.sum(y2 * eqf, axis=1, keepdims=True)
        bx2 = jnp.sum(x2 * eqf, axis=1, keepdims=True)
        yA = jnp.maximum(by1, y1)
        xA = jnp.maximum(bx1, x1)
        yB = jnp.minimum(by2, y2)
        xB = jnp.minimum(bx2, x2)
        inter = jnp.maximum(yB - yA, 0.0) * jnp.maximum(xB - xA, 0.0)
        a1 = (by2 - by1) * (bx2 - bx1)
        iou = inter / (a1 + a2s[...] - inter + 1e-8)
        sref[...] = jnp.where((iou > _IOU_THRES) | eq, _NEG, s)
        valid = best > _NEG * 0.5
        hit = lane128 == k
        ss = jnp.where(hit, jnp.where(valid, best, 0.0), ss)
        sy1 = jnp.where(hit, jnp.where(valid, by1, 0.0), sy1)
        sx1 = jnp.where(hit, jnp.where(valid, bx1, 0.0), sx1)
        sy2 = jnp.where(hit, jnp.where(valid, by2, 0.0), sy2)
        sx2 = jnp.where(hit, jnp.where(valid, bx2, 0.0), sx2)
        return ss, sy1, sx1, sy2, sx2

    init = (jnp.full((RT, 128), -1.0, f32),
            jnp.zeros((RT, 128), f32), jnp.zeros((RT, 128), f32),
            jnp.zeros((RT, 128), f32), jnp.zeros((RT, 128), f32))
    ss, sy1, sx1, sy2, sx2 = jax.lax.fori_loop(0, _MAX_BOX, nms_step, init)
    oss[...] = ss
    osy1[...] = sy1
    osx1[...] = sx1
    osy2[...] = sy2
    osx2[...] = sx2


def _merge_sc_body(C, N,
                   sels_hbm, sy1_hbm, sx1_hbm, sy2_hbm, sx2_hbm,
                   oy1_hbm, ox1_hbm, oy2_hbm, ox2_hbm, osc_hbm, ocl_hbm,
                   ond_hbm,
                   msv, mby1, mbx1, mby2, mbx2,
                   vy1, vx1, vy2, vx2, vsc, vcl, vnd, rotf, roti):
    f32 = jnp.float32
    iota = lax.iota(jnp.int32, _L)
    NBK = C * _KP // _L
    wid = lax.axis_index("s") * 2 + lax.axis_index("c")

    @pl.when(wid < N)
    def _():
        n = wid
        CK = C * _KP
        pltpu.sync_copy(sels_hbm.at[pl.ds(n * CK, CK)], msv)
        pltpu.sync_copy(sy1_hbm.at[pl.ds(n * CK, CK)], mby1)
        pltpu.sync_copy(sx1_hbm.at[pl.ds(n * CK, CK)], mbx1)
        pltpu.sync_copy(sy2_hbm.at[pl.ds(n * CK, CK)], mby2)
        pltpu.sync_copy(sx2_hbm.at[pl.ds(n * CK, CK)], mbx2)

        def mstep(k, carry):
            (lastMv, lastPv, nd,
             ac_s, ac_y1, ac_x1, ac_y2, ac_x2, ac_cl) = carry

            NPB = _KP // _L

            def amax(t, c):
                mx, mp, p1, p2, p3, p4, p5 = c
                sl = pl.ds(t * _L, _L)
                v = msv[sl]
                pos = jnp.broadcast_to(t * _L, (_L,)) + iota
                row = t // NPB
                rowf = jnp.broadcast_to(row, (_L,)).astype(f32)
                elig = (v < lastMv) | ((v == lastMv) & (pos > lastPv))
                take = elig & (v > mx)
                return (jnp.where(take, v, mx), jnp.where(take, pos, mp),
                        jnp.where(take, mby1[sl], p1),
                        jnp.where(take, mbx1[sl], p2),
                        jnp.where(take, mby2[sl], p3),
                        jnp.where(take, mbx2[sl], p4),
                        jnp.where(take, rowf, p5))

            zf = jnp.zeros((_L,), f32)
            mx, mp, p1, p2, p3, p4, p5 = lax.fori_loop(
                0, NBK, amax,
                (jnp.full((_L,), -4.0, f32), jnp.zeros((_L,), jnp.int32),
                 zf, zf, zf, zf, zf))
            Mv, pickv, (by1, bx1, by2, bx2, clsv) = _argmax_bcast(
                mx, mp, [p1, p2, p3, p4, p5], rotf, roti)
            validv = Mv > 0.0
            kmod = k - (k // _L) * _L
            hit = iota == jnp.broadcast_to(kmod, (_L,))
            ac_s2 = jnp.where(hit, Mv, ac_s)
            ac_y12 = jnp.where(hit, jnp.where(validv, by1, 0.0), ac_y1)
            ac_x12 = jnp.where(hit, jnp.where(validv, bx1, 0.0), ac_x1)
            ac_y22 = jnp.where(hit, jnp.where(validv, by2, 0.0), ac_y2)
            ac_x22 = jnp.where(hit, jnp.where(validv, bx2, 0.0), ac_x2)
            ac_cl2 = jnp.where(hit, jnp.where(validv, clsv, 0.0), ac_cl)
            flush = (kmod == _L - 1) | (k == _MAX_BOX - 1)

            @pl.when(flush)
            def _():
                base = k - kmod
                vsc[pl.ds(base, _L)] = ac_s2
                vy1[pl.ds(base, _L)] = ac_y12
                vx1[pl.ds(base, _L)] = ac_x12
                vy2[pl.ds(base, _L)] = ac_y22
                vx2[pl.ds(base, _L)] = ac_x22
                vcl[pl.ds(base, _L)] = ac_cl2

            flvf = jnp.broadcast_to(jnp.where(flush, 1.0, 0.0), (_L,))
            keepf = 1.0 - flvf
            return (Mv, pickv, nd + jnp.where(Mv > 0.0, 1, 0),
                    ac_s2 * keepf,
                    ac_y12 * keepf,
                    ac_x12 * keepf,
                    ac_y22 * keepf,
                    ac_x22 * keepf,
                    ac_cl2 * keepf)

        zfm = jnp.zeros((_L,), f32)
        out_carry = lax.fori_loop(
            0, _MAX_BOX, mstep,
            (jnp.full((_L,), 3.4e38, f32), jnp.full((_L,), -1, jnp.int32),
             jnp.zeros((_L,), jnp.int32), zfm, zfm, zfm, zfm, zfm, zfm))
        nd = out_carry[2]

        vnd[...] = nd
        pltpu.sync_copy(vy1, oy1_hbm.at[pl.ds(n * _KP, _KP)])
        pltpu.sync_copy(vx1, ox1_hbm.at[pl.ds(n * _KP, _KP)])
        pltpu.sync_copy(vy2, oy2_hbm.at[pl.ds(n * _KP, _KP)])
        pltpu.sync_copy(vx2, ox2_hbm.at[pl.ds(n * _KP, _KP)])
        pltpu.sync_copy(vsc, osc_hbm.at[pl.ds(n * _KP, _KP)])
        pltpu.sync_copy(vcl, ocl_hbm.at[pl.ds(n * _KP, _KP)])
        pltpu.sync_copy(vnd, ond_hbm.at[pl.ds(n * _L, _L)])


def kernel(rois, rcnn_conf, rcnn_deltas, input_image):
    N, B = rois.shape[0], rois.shape[1]
    C = rcnn_conf.shape[2] - 1
    H = float(input_image.shape[2])
    W = float(input_image.shape[3])
    R = N * C
    BP = ((B + 6 * _L - 1) // (6 * _L)) * (6 * _L)
    padw = BP - B
    f = jnp.float32

    score_rb = jnp.transpose(rcnn_conf[:, :, :C], (0, 2, 1)).reshape(R, B)
    dd_rb = jnp.transpose(rcnn_deltas.reshape(N, B, C, 4),
                          (0, 2, 3, 1)).reshape(R, 4, B)
    ro_rb = jnp.transpose(rois, (0, 2, 1))

    NW = 32
    RS = min(R, NW)
    RT = R - RS

    score_t = jnp.pad(score_rb, ((0, 0), (0, padw))).reshape(-1)
    ro_t = jnp.pad(ro_rb, ((0, 0), (0, 0), (0, padw))).reshape(-1)
    dl_t = jnp.pad(dd_rb, ((0, 0), (0, 0), (0, padw))).reshape(-1)

    mesh = plsc.VectorSubcoreMesh(core_axis_name="c", subcore_axis_name="s",
                                  num_cores=2, num_subcores=16)

    k1 = pl.kernel(
        functools.partial(_nms_sc_body, C, RS, B, BP, H, W, NW),
        out_type=[jax.ShapeDtypeStruct((RS * _KP,), f) for _ in range(5)],
        mesh=mesh,
        scratch_types=([pltpu.VMEM((BP,), f) for _ in range(14)]
                       + [pltpu.VMEM((_KP,), f) for _ in range(5)]
                       + [pltpu.VMEM((2 * _L,), f),
                          pltpu.VMEM((2 * _L,), jnp.int32)]),
    )
    sc_out = k1(score_t, ro_t, dl_t)

    BPT = ((B + 127) // 128) * 128
    padt = BPT - B

    def pt(x):
        return jnp.pad(x, ((0, 0), (0, padt)))

    rb_t = jnp.broadcast_to(rois[:, None, :, :], (N, C, B, 4))
    rb_t = rb_t.reshape(R, B, 4)[RS:]
    args_tc = [pt(score_rb[RS:]),
               pt(rb_t[:, :, 0]), pt(rb_t[:, :, 1]),
               pt(rb_t[:, :, 2]), pt(rb_t[:, :, 3]),
               pt(dd_rb[RS:, 0, :]), pt(dd_rb[RS:, 1, :]),
               pt(dd_rb[RS:, 2, :]), pt(dd_rb[RS:, 3, :])]
    tc_out = pl.pallas_call(
        functools.partial(_nms_tc_body, H, W, B),
        out_shape=[jax.ShapeDtypeStruct((RT, 128), f) for _ in range(5)],
        scratch_shapes=[pltpu.VMEM((RT, BPT), f) for _ in range(6)],
    )(*args_tc)

    merged_in = [
        jnp.concatenate([a.reshape(RS, _KP), b[:, :_KP]], axis=0).reshape(-1)
        for a, b in zip(sc_out, tc_out)]
    sels, sy1, sx1, sy2, sx2 = merged_in

    k2 = pl.kernel(
        functools.partial(_merge_sc_body, C, N),
        out_type=([jax.ShapeDtypeStruct((N * _KP,), f) for _ in range(6)]
                  + [jax.ShapeDtypeStruct((N * _L,), jnp.int32)]),
        mesh=mesh,
        scratch_types=([pltpu.VMEM((C * _KP,), f) for _ in range(5)]
                       + [pltpu.VMEM((_KP,), f) for _ in range(6)]
                       + [pltpu.VMEM((_L,), jnp.int32)]
                       + [pltpu.VMEM((2 * _L,), f),
                          pltpu.VMEM((2 * _L,), jnp.int32)]),
    )
    oy1, ox1, oy2, ox2, osc, ocl, ond = k2(sels, sy1, sx1, sy2, sx2)

    oy1 = oy1.reshape(N, _KP)
    ox1 = ox1.reshape(N, _KP)
    oy2 = oy2.reshape(N, _KP)
    ox2 = ox2.reshape(N, _KP)
    osc = osc.reshape(N, _KP)
    ocl = ocl.reshape(N, _KP)
    ond = ond.reshape(N, _L)

    nmsed_boxes = jnp.stack(
        [oy1[:, :_MAX_BOX], ox1[:, :_MAX_BOX],
         oy2[:, :_MAX_BOX], ox2[:, :_MAX_BOX]], axis=-1)
    return (nmsed_boxes, osc[:, :_MAX_BOX], ocl[:, :_MAX_BOX], ond[:, 0],
            rois)


# speedup vs baseline: 2.2411x; 2.2411x over previous
import functools

import jax
import jax.numpy as jnp
from jax import lax
from jax.experimental import pallas as pl
from jax.experimental.pallas import tpu as pltpu
from jax.experimental.pallas import tpu_sc as plsc

_MAX_BOX = 100
_IOU_THRES = 0.5
_SCORE_THRES = 0.05
_NEG = -1e10
_L = 16
_KP = 112


def _rot(buf, v, sh):
    buf[pl.ds(0, _L)] = v
    buf[pl.ds(_L, _L)] = v
    return buf[pl.ds(sh, _L)]


def _argmax_bcast(key, idx, payloads, rotf, roti):
    for sh in (8, 4, 2, 1):
        pk = _rot(rotf, key, sh)
        pi = _rot(roti, idx, sh)
        take = (pk > key) | ((pk == key) & (pi < idx))
        key = jnp.where(take, pk, key)
        idx = jnp.where(take, pi, idx)
        payloads = [jnp.where(take, _rot(rotf, p, sh), p)
                    for p in payloads]
    return key, idx, payloads


def _nms_sc_body(C, RS, B, BP, H, W, NW,
                 sc_hbm, ro_hbm, dl_hbm,
                 sels_hbm, sy1_hbm, sx1_hbm, sy2_hbm, sx2_hbm,
                 sv, ry1, rx1, ry2, rx2, d0, d1, d2, d3,
                 y1v, x1v, y2v, x2v, a2v,
                 sel_s, sel_y1, sel_x1, sel_y2, sel_x2, rotf, roti):
    NB = BP // _L
    f32 = jnp.float32
    iota = lax.iota(jnp.int32, _L)
    wid = lax.axis_index("s") * 2 + lax.axis_index("c")

    def do_class(r):
        n = r // C
        pltpu.sync_copy(sc_hbm.at[pl.ds(r * BP, BP)], sv)
        pltpu.sync_copy(ro_hbm.at[pl.ds((n * 4 + 0) * BP, BP)], ry1)
        pltpu.sync_copy(ro_hbm.at[pl.ds((n * 4 + 1) * BP, BP)], rx1)
        pltpu.sync_copy(ro_hbm.at[pl.ds((n * 4 + 2) * BP, BP)], ry2)
        pltpu.sync_copy(ro_hbm.at[pl.ds((n * 4 + 3) * BP, BP)], rx2)
        pltpu.sync_copy(dl_hbm.at[pl.ds((r * 4 + 0) * BP, BP)], d0)
        pltpu.sync_copy(dl_hbm.at[pl.ds((r * 4 + 1) * BP, BP)], d1)
        pltpu.sync_copy(dl_hbm.at[pl.ds((r * 4 + 2) * BP, BP)], d2)
        pltpu.sync_copy(dl_hbm.at[pl.ds((r * 4 + 3) * BP, BP)], d3)

        def dec(i, carry):
            mx, mi, py1, px1, py2, px2 = carry
            sl = pl.ds(i * _L, _L)
            a = ry1[sl]
            b = rx1[sl]
            c2 = ry2[sl]
            e = rx2[sl]
            w0 = e - b + 1.0
            h0 = c2 - a + 1.0
            x0 = b + w0 / 2.0
            y0 = a + h0 / 2.0
            cx = (d0[sl] / 10.0) * w0 + x0
            cy = (d1[sl] / 10.0) * h0 + y0
            wwv = jnp.exp(d2[sl] / 5.0) * w0
            hhv = jnp.exp(d3[sl] / 5.0) * h0
            xx1 = jnp.clip(cx - 0.5 * wwv, 0.0, W - 1.0)
            yy1 = jnp.clip(cy - 0.5 * hhv, 0.0, H - 1.0)
            xx2 = jnp.clip(cx + 0.5 * wwv, 0.0, W - 1.0)
            yy2 = jnp.clip(cy + 0.5 * hhv, 0.0, H - 1.0)
            y1v[sl] = yy1
            x1v[sl] = xx1
            y2v[sl] = yy2
            x2v[sl] = xx2
            a2v[sl] = (yy2 - yy1) * (xx2 - xx1)
            s = sv[sl]
            gidx = jnp.broadcast_to(i * _L, (_L,)) + iota
            keep = (s > _SCORE_THRES) & (gidx < B)
            snew = jnp.where(keep, s, _NEG)
            sv[sl] = snew
            take = snew > mx
            return (jnp.where(take, snew, mx), jnp.where(take, gidx, mi),
                    jnp.where(take, yy1, py1), jnp.where(take, xx1, px1),
                    jnp.where(take, yy2, py2), jnp.where(take, xx2, px2))

        zf0 = jnp.zeros((_L,), f32)
        mx, mi, py1, px1, py2, px2 = lax.fori_loop(
            0, NB, dec,
            (jnp.full((_L,), -2e10, f32), jnp.zeros((_L,), jnp.int32),
             zf0, zf0, zf0, zf0))
        Mv0, giv0, (by10, bx10, by20, bx20) = _argmax_bcast(
            mx, mi, [py1, px1, py2, px2], rotf, roti)

        def step(k, carry):
            (Mv, giv, by1, bx1, by2, bx2,
             acc_s, acc_y1, acc_x1, acc_y2, acc_x2) = carry

            validv = Mv > (_NEG * 0.5)
            kmod = k - (k // _L) * _L
            hit = iota == jnp.broadcast_to(kmod, (_L,))
            acc_s2 = jnp.where(hit, jnp.where(validv, Mv, 0.0), acc_s)
            acc_y12 = jnp.where(hit, jnp.where(validv, by1, 0.0), acc_y1)
            acc_x12 = jnp.where(hit, jnp.where(validv, bx1, 0.0), acc_x1)
            acc_y22 = jnp.where(hit, jnp.where(validv, by2, 0.0), acc_y2)
            acc_x22 = jnp.where(hit, jnp.where(validv, bx2, 0.0), acc_x2)
            flush = (kmod == _L - 1) | (k == _MAX_BOX - 1)

            @pl.when(flush)
            def _():
                base = k - kmod
                sel_s[pl.ds(base, _L)] = acc_s2
                sel_y1[pl.ds(base, _L)] = acc_y12
                sel_x1[pl.ds(base, _L)] = acc_x12
                sel_y2[pl.ds(base, _L)] = acc_y22
                sel_x2[pl.ds(base, _L)] = acc_x22

            a1 = (by2 - by1) * (bx2 - bx1)

            def sweep4(i, c):
                slots = []
                for u in range(4):
                    mx, mi, py1, px1, py2, px2 = c[6 * u:6 * u + 6]
                    blk = i * 4 + u
                    sl = pl.ds(blk * _L, _L)
                    y1 = y1v[sl]
                    x1 = x1v[sl]
                    y2 = y2v[sl]
                    x2 = x2v[sl]
                    yA = jnp.maximum(by1, y1)
                    xA = jnp.maximum(bx1, x1)
                    yB = jnp.minimum(by2, y2)
                    xB = jnp.minimum(bx2, x2)
                    inter = (jnp.maximum(yB - yA, 0.0)
                             * jnp.maximum(xB - xA, 0.0))
                    iou = inter / (a1 + a2v[sl] - inter + 1e-8)
                    gidx = jnp.broadcast_to(blk * _L, (_L,)) + iota
                    s = sv[sl]
                    snew = jnp.where((iou > _IOU_THRES) | (gidx == giv),
                                     _NEG, s)
                    sv[sl] = snew
                    take = snew > mx
                    slots.extend(
                        (jnp.where(take, snew, mx),
                         jnp.where(take, gidx, mi),
                         jnp.where(take, y1, py1),
                         jnp.where(take, x1, px1),
                         jnp.where(take, y2, py2),
                         jnp.where(take, x2, px2)))
                return tuple(slots)

            zf = jnp.zeros((_L,), f32)
            init1 = (jnp.full((_L,), -2e10, f32),
                     jnp.zeros((_L,), jnp.int32), zf, zf, zf, zf)
            out4 = lax.fori_loop(0, NB // 4, sweep4, init1 * 4)

            def comb(sa, sb):
                take = ((sb[0] > sa[0])
                        | ((sb[0] == sa[0]) & (sb[1] < sa[1])))
                return tuple(jnp.where(take, b, a)
                             for a, b in zip(sa, sb))

            s01 = comb(out4[0:6], out4[6:12])
            s23 = comb(out4[12:18], out4[18:24])
            mx, mi, py1, px1, py2, px2 = comb(s01, s23)
            nMv, ngiv, (nby1, nbx1, nby2, nbx2) = _argmax_bcast(
                mx, mi, [py1, px1, py2, px2], rotf, roti)

            flvf = jnp.broadcast_to(jnp.where(flush, 1.0, 0.0), (_L,))
            keepf = 1.0 - flvf
            return (nMv, ngiv, nby1, nbx1, nby2, nbx2,
                    acc_s2 * keepf - flvf,
                    acc_y12 * keepf,
                    acc_x12 * keepf,
                    acc_y22 * keepf,
                    acc_x22 * keepf)

        lax.fori_loop(0, _MAX_BOX, step,
                      (Mv0, giv0, by10, bx10, by20, bx20,
                       jnp.full((_L,), -1.0, f32), zf0, zf0, zf0, zf0))

        pltpu.sync_copy(sel_s, sels_hbm.at[pl.ds(r * _KP, _KP)])
        pltpu.sync_copy(sel_y1, sy1_hbm.at[pl.ds(r * _KP, _KP)])
        pltpu.sync_copy(sel_x1, sx1_hbm.at[pl.ds(r * _KP, _KP)])
        pltpu.sync_copy(sel_y2, sy2_hbm.at[pl.ds(r * _KP, _KP)])
        pltpu.sync_copy(sel_x2, sx2_hbm.at[pl.ds(r * _KP, _KP)])

    for j in range(-(-RS // NW)):
        r = wid + NW * j

        @pl.when(r < RS)
        def _():
            do_class(r)


def _nms_sc_pair_body(C, RS, B, BP, H, W,
                      sc_hbm, ro_hbm, dl_hbm,
                      sels_hbm, sy1_hbm, sx1_hbm, sy2_hbm, sx2_hbm,
                      sv, ry1, rx1, ry2, rx2, d0, d1, d2, d3,
                      y1v, x1v, y2v, x2v, a2v,
                      sel_s, sel_y1, sel_x1, sel_y2, sel_x2,
                      rotf, roti, xf, xf2, xi, xi2, shf, shi):
    HB = BP // 2
    NBH = HB // _L
    f32 = jnp.float32
    iota = lax.iota(jnp.int32, _L)
    c = lax.axis_index("c")
    s = lax.axis_index("s")
    h = s - (s // 2) * 2
    r = c * (RS // 2) + s // 2
    n = r // C
    base = h * HB

    pltpu.sync_copy(sc_hbm.at[pl.ds(r * BP + base, HB)], sv)
    pltpu.sync_copy(ro_hbm.at[pl.ds((n * 4 + 0) * BP + base, HB)], ry1)
    pltpu.sync_copy(ro_hbm.at[pl.ds((n * 4 + 1) * BP + base, HB)], rx1)
    pltpu.sync_copy(ro_hbm.at[pl.ds((n * 4 + 2) * BP + base, HB)], ry2)
    pltpu.sync_copy(ro_hbm.at[pl.ds((n * 4 + 3) * BP + base, HB)], rx2)
    pltpu.sync_copy(dl_hbm.at[pl.ds((r * 4 + 0) * BP + base, HB)], d0)
    pltpu.sync_copy(dl_hbm.at[pl.ds((r * 4 + 1) * BP + base, HB)], d1)
    pltpu.sync_copy(dl_hbm.at[pl.ds((r * 4 + 2) * BP + base, HB)], d2)
    pltpu.sync_copy(dl_hbm.at[pl.ds((r * 4 + 3) * BP + base, HB)], d3)

    def exchange(kk, Mv, giv, by1, bx1, by2, bx2):
        buf = kk - (kk // 2) * 2
        xf[pl.ds(0, _L)] = Mv
        xf[pl.ds(_L, _L)] = by1
        xf[pl.ds(2 * _L, _L)] = bx1
        xf[pl.ds(3 * _L, _L)] = by2
        xf[pl.ds(4 * _L, _L)] = bx2
        xi[pl.ds(0, _L)] = giv
        slot = buf * 16 + s
        pltpu.sync_copy(xf, shf.at[pl.ds(slot * 5 * _L, 5 * _L)])
        pltpu.sync_copy(xi, shi.at[pl.ds(slot * _L, _L)])
        plsc.subcore_barrier()
        pslot = buf * 16 + jnp.bitwise_xor(s, 1)
        pltpu.sync_copy(shf.at[pl.ds(pslot * 5 * _L, 5 * _L)], xf2)
        pltpu.sync_copy(shi.at[pl.ds(pslot * _L, _L)], xi2)
        pMv = xf2[pl.ds(0, _L)]
        pby1 = xf2[pl.ds(_L, _L)]
        pbx1 = xf2[pl.ds(2 * _L, _L)]
        pby2 = xf2[pl.ds(3 * _L, _L)]
        pbx2 = xf2[pl.ds(4 * _L, _L)]
        pgiv = xi2[pl.ds(0, _L)]
        take = (pMv > Mv) | ((pMv == Mv) & (pgiv < giv))
        return (jnp.where(take, pMv, Mv), jnp.where(take, pgiv, giv),
                jnp.where(take, pby1, by1), jnp.where(take, pbx1, bx1),
                jnp.where(take, pby2, by2), jnp.where(take, pbx2, bx2))

    def dec(i, carry):
        mx, mi, py1, px1, py2, px2 = carry
        sl = pl.ds(i * _L, _L)
        a = ry1[sl]
        b = rx1[sl]
        c2 = ry2[sl]
        e = rx2[sl]
        w0 = e - b + 1.0
        h0 = c2 - a + 1.0
        x0 = b + w0 / 2.0
        y0 = a + h0 / 2.0
        cx = (d0[sl] / 10.0) * w0 + x0
        cy = (d1[sl] / 10.0) * h0 + y0
        wwv = jnp.exp(d2[sl] / 5.0) * w0
        hhv = jnp.exp(d3[sl] / 5.0) * h0
        xx1 = jnp.clip(cx - 0.5 * wwv, 0.0, W - 1.0)
        yy1 = jnp.clip(cy - 0.5 * hhv, 0.0, H - 1.0)
        xx2 = jnp.clip(cx + 0.5 * wwv, 0.0, W - 1.0)
        yy2 = jnp.clip(cy + 0.5 * hhv, 0.0, H - 1.0)
        y1v[sl] = yy1
        x1v[sl] = xx1
        y2v[sl] = yy2
        x2v[sl] = xx2
        a2v[sl] = (yy2 - yy1) * (xx2 - xx1)
        sc = sv[sl]
        gidx = jnp.broadcast_to(base + i * _L, (_L,)) + iota
        keep = (sc > _SCORE_THRES) & (gidx < B)
        snew = jnp.where(keep, sc, _NEG)
        sv[sl] = snew
        take = snew > mx
        return (jnp.where(take, snew, mx), jnp.where(take, gidx, mi),
                jnp.where(take, yy1, py1), jnp.where(take, xx1, px1),
                jnp.where(take, yy2, py2), jnp.where(take, xx2, px2))

    zf0 = jnp.zeros((_L,), f32)
    mx, mi, py1, px1, py2, px2 = lax.fori_loop(
        0, NBH, dec,
        (jnp.full((_L,), -2e10, f32), jnp.zeros((_L,), jnp.int32),
         zf0, zf0, zf0, zf0))
    Mv0, giv0, (by10, bx10, by20, bx20) = _argmax_bcast(
        mx, mi, [py1, px1, py2, px2], rotf, roti)
    Mv0, giv0, by10, bx10, by20, bx20 = exchange(
        0, Mv0, giv0, by10, bx10, by20, bx20)

    def step(k, carry):
        (Mv, giv, by1, bx1, by2, bx2,
         acc_s, acc_y1, acc_x1, acc_y2, acc_x2) = carry

        validv = Mv > (_NEG * 0.5)
        kmod = k - (k // _L) * _L
        hit = iota == jnp.broadcast_to(kmod, (_L,))
        acc_s2 = jnp.where(hit, jnp.where(validv, Mv, 0.0), acc_s)
        acc_y12 = jnp.where(hit, jnp.where(validv, by1, 0.0), acc_y1)
        acc_x12 = jnp.where(hit, jnp.where(validv, bx1, 0.0), acc_x1)
        acc_y22 = jnp.where(hit, jnp.where(validv, by2, 0.0), acc_y2)
        acc_x22 = jnp.where(hit, jnp.where(validv, bx2, 0.0), acc_x2)
        flush = (kmod == _L - 1) | (k == _MAX_BOX - 1)

        @pl.when(flush)
        def _():
            fbase = k - kmod
            sel_s[pl.ds(fbase, _L)] = acc_s2
            sel_y1[pl.ds(fbase, _L)] = acc_y12
            sel_x1[pl.ds(fbase, _L)] = acc_x12
            sel_y2[pl.ds(fbase, _L)] = acc_y22
            sel_x2[pl.ds(fbase, _L)] = acc_x22

        a1 = (by2 - by1) * (bx2 - bx1)

        def sweep4(i, cc):
            slots = []
            for u in range(4):
                mx, mi, py1, px1, py2, px2 = cc[6 * u:6 * u + 6]
                blk = i * 4 + u
                sl = pl.ds(blk * _L, _L)
                y1 = y1v[sl]
                x1 = x1v[sl]
                y2 = y2v[sl]
                x2 = x2v[sl]
                yA = jnp.maximum(by1, y1)
                xA = jnp.maximum(bx1, x1)
                yB = jnp.minimum(by2, y2)
                xB = jnp.minimum(bx2, x2)
                inter = (jnp.maximum(yB - yA, 0.0)
                         * jnp.maximum(xB - xA, 0.0))
                iou = inter / (a1 + a2v[sl] - inter + 1e-8)
                gidx = jnp.broadcast_to(base + blk * _L, (_L,)) + iota
                sc = sv[sl]
                snew = jnp.where((iou > _IOU_THRES) | (gidx == giv),
                                 _NEG, sc)
                sv[sl] = snew
                take = snew > mx
                slots.extend(
                    (jnp.where(take, snew, mx),
                     jnp.where(take, gidx, mi),
                     jnp.where(take, y1, py1),
                     jnp.where(take, x1, px1),
                     jnp.where(take, y2, py2),
                     jnp.where(take, x2, px2)))
            return tuple(slots)

        zf = jnp.zeros((_L,), f32)
        init1 = (jnp.full((_L,), -2e10, f32),
                 jnp.zeros((_L,), jnp.int32), zf, zf, zf, zf)
        out4 = lax.fori_loop(0, NBH // 4, sweep4, init1 * 4)

        def comb(sa, sb):
            take = ((sb[0] > sa[0])
                    | ((sb[0] == sa[0]) & (sb[1] < sa[1])))
            return tuple(jnp.where(take, b, a) for a, b in zip(sa, sb))

        s01 = comb(out4[0:6], out4[6:12])
        s23 = comb(out4[12:18], out4[18:24])
        mx, mi, py1, px1, py2, px2 = comb(s01, s23)
        nMv, ngiv, (nby1, nbx1, nby2, nbx2) = _argmax_bcast(
            mx, mi, [py1, px1, py2, px2], rotf, roti)
        nMv, ngiv, nby1, nbx1, nby2, nbx2 = exchange(
            k + 1, nMv, ngiv, nby1, nbx1, nby2, nbx2)

        flvf = jnp.broadcast_to(jnp.where(flush, 1.0, 0.0), (_L,))
        keepf = 1.0 - flvf
        return (nMv, ngiv, nby1, nbx1, nby2, nbx2,
                acc_s2 * keepf - flvf,
                acc_y12 * keepf,
                acc_x12 * keepf,
                acc_y22 * keepf,
                acc_x22 * keepf)

    lax.fori_loop(0, _MAX_BOX, step,
                  (Mv0, giv0, by10, bx10, by20, bx20,
                   jnp.full((_L,), -1.0, f32), zf0, zf0, zf0, zf0))

    @pl.when(h == 0)
    def _():
        pltpu.sync_copy(sel_s, sels_hbm.at[pl.ds(r * _KP, _KP)])
        pltpu.sync_copy(sel_y1, sy1_hbm.at[pl.ds(r * _KP, _KP)])
        pltpu.sync_copy(sel_x1, sx1_hbm.at[pl.ds(r * _KP, _KP)])
        pltpu.sync_copy(sel_y2, sy2_hbm.at[pl.ds(r * _KP, _KP)])
        pltpu.sync_copy(sel_x2, sx2_hbm.at[pl.ds(r * _KP, _KP)])


def _nms_tc_body(H, W, B,
                 s_in, ry1, rx1, ry2, rx2, t0, t1, t2, t3,
                 oss, osy1, osx1, osy2, osx2,
                 sref, y1s, x1s, y2s, x2s, a2s):
    RT, BPT = s_in.shape
    f32 = jnp.float32

    w0 = rx2[...] - rx1[...] + 1.0
    h0 = ry2[...] - ry1[...] + 1.0
    x0 = rx1[...] + w0 / 2.0
    y0 = ry1[...] + h0 / 2.0
    cx = (t0[...] / 10.0) * w0 + x0
    cy = (t1[...] / 10.0) * h0 + y0
    ww = jnp.exp(t2[...] / 5.0) * w0
    hh = jnp.exp(t3[...] / 5.0) * h0
    xx1 = jnp.clip(cx - 0.5 * ww, 0.0, W - 1.0)
    yy1 = jnp.clip(cy - 0.5 * hh, 0.0, H - 1.0)
    xx2 = jnp.clip(cx + 0.5 * ww, 0.0, W - 1.0)
    yy2 = jnp.clip(cy + 0.5 * hh, 0.0, H - 1.0)
    y1s[...] = yy1
    x1s[...] = xx1
    y2s[...] = yy2
    x2s[...] = xx2
    a2s[...] = (yy2 - yy1) * (xx2 - xx1)

    lane = jax.lax.broadcasted_iota(jnp.int32, (RT, BPT), 1)
    s = s_in[...]
    sref[...] = jnp.where((lane < B) & (s > _SCORE_THRES), s, _NEG)

    lane128 = jax.lax.broadcasted_iota(jnp.int32, (RT, 128), 1)

    def nms_step(k, carry):
        ss, sy1, sx1, sy2, sx2 = carry
        s = sref[...]
        best = jnp.max(s, axis=1, keepdims=True)
        idx = jnp.min(jnp.where(s == best, lane, BPT), axis=1,
                      keepdims=True)
        eq = lane == idx
        eqf = eq.astype(f32)
        y1 = y1s[...]
        x1 = x1s[...]
        y2 = y2s[...]
        x2 = x2s[...]
        by1 = jnp.sum(y1 * eqf, axis=1, keepdims=True)
        bx1 = jnp.sum(x1 * eqf, axis=1, keepdims=True)
        by2 = jnp.sum(y2 * eqf, axis=1, keepdims=True)
        bx2 = jnp.sum(x2 * eqf, axis=1, keepdims=True)
        yA = jnp.maximum(by1, y1)
        xA = jnp.maximum(bx1, x1)
        yB = jnp.minimum(by2, y2)
        xB = jnp.minimum(bx2, x2)
        inter = jnp.maximum(yB - yA, 0.0) * jnp.maximum(xB - xA, 0.0)
        a1 = (by2 - by1) * (bx2 - bx1)
        iou = inter / (a1 + a2s[...] - inter + 1e-8)
        sref[...] = jnp.where((iou > _IOU_THRES) | eq, _NEG, s)
        valid = best > _NEG * 0.5
        hit = lane128 == k
        ss = jnp.where(hit, jnp.where(valid, best, 0.0), ss)
        sy1 = jnp.where(hit, jnp.where(valid, by1, 0.0), sy1)
        sx1 = jnp.where(hit, jnp.where(valid, bx1, 0.0), sx1)
        sy2 = jnp.where(hit, jnp.where(valid, by2, 0.0), sy2)
        sx2 = jnp.where(hit, jnp.where(valid, bx2, 0.0), sx2)
        return ss, sy1, sx1, sy2, sx2

    init = (jnp.full((RT, 128), -1.0, f32),
            jnp.zeros((RT, 128), f32), jnp.zeros((RT, 128), f32),
            jnp.zeros((RT, 128), f32), jnp.zeros((RT, 128), f32))
    ss, sy1, sx1, sy2, sx2 = jax.lax.fori_loop(0, _MAX_BOX, nms_step, init)
    oss[...] = ss
    osy1[...] = sy1
    osx1[...] = sx1
    osy2[...] = sy2
    osx2[...] = sx2


def _merge_sc_body(C, N,
                   sels_hbm, sy1_hbm, sx1_hbm, sy2_hbm, sx2_hbm,
                   oy1_hbm, ox1_hbm, oy2_hbm, ox2_hbm, osc_hbm, ocl_hbm,
                   ond_hbm,
                   msv, mby1, mbx1, mby2, mbx2,
                   vy1, vx1, vy2, vx2, vsc, vcl, vnd, rotf, roti):
    f32 = jnp.float32
    iota = lax.iota(jnp.int32, _L)
    NBK = C * _KP // _L
    wid = lax.axis_index("s") * 2 + lax.axis_index("c")

    @pl.when(wid < N)
    def _():
        n = wid
        CK = C * _KP
        pltpu.sync_copy(sels_hbm.at[pl.ds(n * CK, CK)], msv)
        pltpu.sync_copy(sy1_hbm.at[pl.ds(n * CK, CK)], mby1)
        pltpu.sync_copy(sx1_hbm.at[pl.ds(n * CK, CK)], mbx1)
        pltpu.sync_copy(sy2_hbm.at[pl.ds(n * CK, CK)], mby2)
        pltpu.sync_copy(sx2_hbm.at[pl.ds(n * CK, CK)], mbx2)

        def mstep(k, carry):
            (lastMv, lastPv, nd,
             ac_s, ac_y1, ac_x1, ac_y2, ac_x2, ac_cl) = carry

            NPB = _KP // _L

            def amax(t, c):
                mx, mp, p1, p2, p3, p4, p5 = c
                sl = pl.ds(t * _L, _L)
                v = msv[sl]
                pos = jnp.broadcast_to(t * _L, (_L,)) + iota
                row = t // NPB
                rowf = jnp.broadcast_to(row, (_L,)).astype(f32)
                elig = (v < lastMv) | ((v == lastMv) & (pos > lastPv))
                take = elig & (v > mx)
                return (jnp.where(take, v, mx), jnp.where(take, pos, mp),
                        jnp.where(take, mby1[sl], p1),
                        jnp.where(take, mbx1[sl], p2),
                        jnp.where(take, mby2[sl], p3),
                        jnp.where(take, mbx2[sl], p4),
                        jnp.where(take, rowf, p5))

            zf = jnp.zeros((_L,), f32)
            mx, mp, p1, p2, p3, p4, p5 = lax.fori_loop(
                0, NBK, amax,
                (jnp.full((_L,), -4.0, f32), jnp.zeros((_L,), jnp.int32),
                 zf, zf, zf, zf, zf))
            Mv, pickv, (by1, bx1, by2, bx2, clsv) = _argmax_bcast(
                mx, mp, [p1, p2, p3, p4, p5], rotf, roti)
            validv = Mv > 0.0
            kmod = k - (k // _L) * _L
            hit = iota == jnp.broadcast_to(kmod, (_L,))
            ac_s2 = jnp.where(hit, Mv, ac_s)
            ac_y12 = jnp.where(hit, jnp.where(validv, by1, 0.0), ac_y1)
            ac_x12 = jnp.where(hit, jnp.where(validv, bx1, 0.0), ac_x1)
            ac_y22 = jnp.where(hit, jnp.where(validv, by2, 0.0), ac_y2)
            ac_x22 = jnp.where(hit, jnp.where(validv, bx2, 0.0), ac_x2)
            ac_cl2 = jnp.where(hit, jnp.where(validv, clsv, 0.0), ac_cl)
            flush = (kmod == _L - 1) | (k == _MAX_BOX - 1)

            @pl.when(flush)
            def _():
                base = k - kmod
                vsc[pl.ds(base, _L)] = ac_s2
                vy1[pl.ds(base, _L)] = ac_y12
                vx1[pl.ds(base, _L)] = ac_x12
                vy2[pl.ds(base, _L)] = ac_y22
                vx2[pl.ds(base, _L)] = ac_x22
                vcl[pl.ds(base, _L)] = ac_cl2

            flvf = jnp.broadcast_to(jnp.where(flush, 1.0, 0.0), (_L,))
            keepf = 1.0 - flvf
            return (Mv, pickv, nd + jnp.where(Mv > 0.0, 1, 0),
                    ac_s2 * keepf,
                    ac_y12 * keepf,
                    ac_x12 * keepf,
                    ac_y22 * keepf,
                    ac_x22 * keepf,
                    ac_cl2 * keepf)

        zfm = jnp.zeros((_L,), f32)
        out_carry = lax.fori_loop(
            0, _MAX_BOX, mstep,
            (jnp.full((_L,), 3.4e38, f32), jnp.full((_L,), -1, jnp.int32),
             jnp.zeros((_L,), jnp.int32), zfm, zfm, zfm, zfm, zfm, zfm))
        nd = out_carry[2]

        vnd[...] = nd
        pltpu.sync_copy(vy1, oy1_hbm.at[pl.ds(n * _KP, _KP)])
        pltpu.sync_copy(vx1, ox1_hbm.at[pl.ds(n * _KP, _KP)])
        pltpu.sync_copy(vy2, oy2_hbm.at[pl.ds(n * _KP, _KP)])
        pltpu.sync_copy(vx2, ox2_hbm.at[pl.ds(n * _KP, _KP)])
        pltpu.sync_copy(vsc, osc_hbm.at[pl.ds(n * _KP, _KP)])
        pltpu.sync_copy(vcl, ocl_hbm.at[pl.ds(n * _KP, _KP)])
        pltpu.sync_copy(vnd, ond_hbm.at[pl.ds(n * _L, _L)])


def kernel(rois, rcnn_conf, rcnn_deltas, input_image):
    N, B = rois.shape[0], rois.shape[1]
    C = rcnn_conf.shape[2] - 1
    H = float(input_image.shape[2])
    W = float(input_image.shape[3])
    R = N * C
    BP = ((B + 8 * _L - 1) // (8 * _L)) * (8 * _L)
    padw = BP - B
    f = jnp.float32

    score_rb = jnp.transpose(rcnn_conf[:, :, :C], (0, 2, 1)).reshape(R, B)
    dd_rb = jnp.transpose(rcnn_deltas.reshape(N, B, C, 4),
                          (0, 2, 3, 1)).reshape(R, 4, B)
    ro_rb = jnp.transpose(rois, (0, 2, 1))

    NW = 32
    RS = min(R, NW // 2)
    RT = R - RS

    score_t = jnp.pad(score_rb, ((0, 0), (0, padw))).reshape(-1)
    ro_t = jnp.pad(ro_rb, ((0, 0), (0, 0), (0, padw))).reshape(-1)
    dl_t = jnp.pad(dd_rb, ((0, 0), (0, 0), (0, padw))).reshape(-1)

    mesh = plsc.VectorSubcoreMesh(core_axis_name="c", subcore_axis_name="s",
                                  num_cores=2, num_subcores=16)

    k1 = pl.kernel(
        functools.partial(_nms_sc_pair_body, C, RS, B, BP, H, W),
        out_type=[jax.ShapeDtypeStruct((RS * _KP,), f) for _ in range(5)],
        mesh=mesh,
        scratch_types=([pltpu.VMEM((BP // 2,), f) for _ in range(14)]
                       + [pltpu.VMEM((_KP,), f) for _ in range(5)]
                       + [pltpu.VMEM((2 * _L,), f),
                          pltpu.VMEM((2 * _L,), jnp.int32)]
                       + [pltpu.VMEM((5 * _L,), f), pltpu.VMEM((5 * _L,), f),
                          pltpu.VMEM((_L,), jnp.int32),
                          pltpu.VMEM((_L,), jnp.int32),
                          pltpu.VMEM_SHARED((2 * 16 * 5 * _L,), f),
                          pltpu.VMEM_SHARED((2 * 16 * _L,), jnp.int32)]),
    )
    sc_out = k1(score_t, ro_t, dl_t)

    BPT = ((B + 127) // 128) * 128
    padt = BPT - B

    def pt(x):
        return jnp.pad(x, ((0, 0), (0, padt)))

    rb_t = jnp.broadcast_to(rois[:, None, :, :], (N, C, B, 4))
    rb_t = rb_t.reshape(R, B, 4)[RS:]
    args_tc = [pt(score_rb[RS:]),
               pt(rb_t[:, :, 0]), pt(rb_t[:, :, 1]),
               pt(rb_t[:, :, 2]), pt(rb_t[:, :, 3]),
               pt(dd_rb[RS:, 0, :]), pt(dd_rb[RS:, 1, :]),
               pt(dd_rb[RS:, 2, :]), pt(dd_rb[RS:, 3, :])]
    tc_out = pl.pallas_call(
        functools.partial(_nms_tc_body, H, W, B),
        out_shape=[jax.ShapeDtypeStruct((RT, 128), f) for _ in range(5)],
        scratch_shapes=[pltpu.VMEM((RT, BPT), f) for _ in range(6)],
    )(*args_tc)

    merged_in = [
        jnp.concatenate([a.reshape(RS, _KP), b[:, :_KP]], axis=0).reshape(-1)
        for a, b in zip(sc_out, tc_out)]
    sels, sy1, sx1, sy2, sx2 = merged_in

    k2 = pl.kernel(
        functools.partial(_merge_sc_body, C, N),
        out_type=([jax.ShapeDtypeStruct((N * _KP,), f) for _ in range(6)]
                  + [jax.ShapeDtypeStruct((N * _L,), jnp.int32)]),
        mesh=mesh,
        scratch_types=([pltpu.VMEM((C * _KP,), f) for _ in range(5)]
                       + [pltpu.VMEM((_KP,), f) for _ in range(6)]
                       + [pltpu.VMEM((_L,), jnp.int32)]
                       + [pltpu.VMEM((2 * _L,), f),
                          pltpu.VMEM((2 * _L,), jnp.int32)]),
    )
    oy1, ox1, oy2, ox2, osc, ocl, ond = k2(sels, sy1, sx1, sy2, sx2)

    oy1 = oy1.reshape(N, _KP)
    ox1 = ox1.reshape(N, _KP)
    oy2 = oy2.reshape(N, _KP)
    ox2 = ox2.reshape(N, _KP)
    osc = osc.reshape(N, _KP)
    ocl = ocl.reshape(N, _KP)
    ond = ond.reshape(N, _L)

    nmsed_boxes = jnp.stack(
        [oy1[:, :_MAX_BOX], ox1[:, :_MAX_BOX],
         oy2[:, :_MAX_BOX], ox2[:, :_MAX_BOX]], axis=-1)
    return (nmsed_boxes, osc[:, :_MAX_BOX], ocl[:, :_MAX_BOX], ond[:, 0],
            rois)


# speedup vs baseline: 2.6716x; 1.1921x over previous
import functools

import jax
import jax.numpy as jnp
from jax import lax
from jax.experimental import pallas as pl
from jax.experimental.pallas import tpu as pltpu
from jax.experimental.pallas import tpu_sc as plsc

_MAX_BOX = 100
_IOU_THRES = 0.5
_SCORE_THRES = 0.05
_NEG = -1e10
_L = 16
_KP = 112


def _rot(buf, v, sh):
    buf[pl.ds(0, _L)] = v
    buf[pl.ds(_L, _L)] = v
    return buf[pl.ds(sh, _L)]


def _argmax_bcast(key, idx, payloads, rotf, roti):
    for sh in (8, 4, 2, 1):
        pk = _rot(rotf, key, sh)
        pi = _rot(roti, idx, sh)
        take = (pk > key) | ((pk == key) & (pi < idx))
        key = jnp.where(take, pk, key)
        idx = jnp.where(take, pi, idx)
        payloads = [jnp.where(take, _rot(rotf, p, sh), p)
                    for p in payloads]
    return key, idx, payloads


def _nms_sc_body(C, RS, B, BP, H, W, NW,
                 sc_hbm, ro_hbm, dl_hbm,
                 sels_hbm, sy1_hbm, sx1_hbm, sy2_hbm, sx2_hbm,
                 sv, ry1, rx1, ry2, rx2, d0, d1, d2, d3,
                 y1v, x1v, y2v, x2v, a2v,
                 sel_s, sel_y1, sel_x1, sel_y2, sel_x2, rotf, roti):
    NB = BP // _L
    f32 = jnp.float32
    iota = lax.iota(jnp.int32, _L)
    wid = lax.axis_index("s") * 2 + lax.axis_index("c")

    def do_class(r):
        n = r // C
        pltpu.sync_copy(sc_hbm.at[pl.ds(r * BP, BP)], sv)
        pltpu.sync_copy(ro_hbm.at[pl.ds((n * 4 + 0) * BP, BP)], ry1)
        pltpu.sync_copy(ro_hbm.at[pl.ds((n * 4 + 1) * BP, BP)], rx1)
        pltpu.sync_copy(ro_hbm.at[pl.ds((n * 4 + 2) * BP, BP)], ry2)
        pltpu.sync_copy(ro_hbm.at[pl.ds((n * 4 + 3) * BP, BP)], rx2)
        pltpu.sync_copy(dl_hbm.at[pl.ds((r * 4 + 0) * BP, BP)], d0)
        pltpu.sync_copy(dl_hbm.at[pl.ds((r * 4 + 1) * BP, BP)], d1)
        pltpu.sync_copy(dl_hbm.at[pl.ds((r * 4 + 2) * BP, BP)], d2)
        pltpu.sync_copy(dl_hbm.at[pl.ds((r * 4 + 3) * BP, BP)], d3)

        def dec(i, carry):
            mx, mi, py1, px1, py2, px2 = carry
            sl = pl.ds(i * _L, _L)
            a = ry1[sl]
            b = rx1[sl]
            c2 = ry2[sl]
            e = rx2[sl]
            w0 = e - b + 1.0
            h0 = c2 - a + 1.0
            x0 = b + w0 / 2.0
            y0 = a + h0 / 2.0
            cx = (d0[sl] / 10.0) * w0 + x0
            cy = (d1[sl] / 10.0) * h0 + y0
            wwv = jnp.exp(d2[sl] / 5.0) * w0
            hhv = jnp.exp(d3[sl] / 5.0) * h0
            xx1 = jnp.clip(cx - 0.5 * wwv, 0.0, W - 1.0)
            yy1 = jnp.clip(cy - 0.5 * hhv, 0.0, H - 1.0)
            xx2 = jnp.clip(cx + 0.5 * wwv, 0.0, W - 1.0)
            yy2 = jnp.clip(cy + 0.5 * hhv, 0.0, H - 1.0)
            y1v[sl] = yy1
            x1v[sl] = xx1
            y2v[sl] = yy2
            x2v[sl] = xx2
            a2v[sl] = (yy2 - yy1) * (xx2 - xx1)
            s = sv[sl]
            gidx = jnp.broadcast_to(i * _L, (_L,)) + iota
            keep = (s > _SCORE_THRES) & (gidx < B)
            snew = jnp.where(keep, s, _NEG)
            sv[sl] = snew
            take = snew > mx
            return (jnp.where(take, snew, mx), jnp.where(take, gidx, mi),
                    jnp.where(take, yy1, py1), jnp.where(take, xx1, px1),
                    jnp.where(take, yy2, py2), jnp.where(take, xx2, px2))

        zf0 = jnp.zeros((_L,), f32)
        mx, mi, py1, px1, py2, px2 = lax.fori_loop(
            0, NB, dec,
            (jnp.full((_L,), -2e10, f32), jnp.zeros((_L,), jnp.int32),
             zf0, zf0, zf0, zf0))
        Mv0, giv0, (by10, bx10, by20, bx20) = _argmax_bcast(
            mx, mi, [py1, px1, py2, px2], rotf, roti)

        def step(k, carry):
            (Mv, giv, by1, bx1, by2, bx2,
             acc_s, acc_y1, acc_x1, acc_y2, acc_x2) = carry

            validv = Mv > (_NEG * 0.5)
            kmod = k - (k // _L) * _L
            hit = iota == jnp.broadcast_to(kmod, (_L,))
            acc_s2 = jnp.where(hit, jnp.where(validv, Mv, 0.0), acc_s)
            acc_y12 = jnp.where(hit, jnp.where(validv, by1, 0.0), acc_y1)
            acc_x12 = jnp.where(hit, jnp.where(validv, bx1, 0.0), acc_x1)
            acc_y22 = jnp.where(hit, jnp.where(validv, by2, 0.0), acc_y2)
            acc_x22 = jnp.where(hit, jnp.where(validv, bx2, 0.0), acc_x2)
            flush = (kmod == _L - 1) | (k == _MAX_BOX - 1)

            @pl.when(flush)
            def _():
                base = k - kmod
                sel_s[pl.ds(base, _L)] = acc_s2
                sel_y1[pl.ds(base, _L)] = acc_y12
                sel_x1[pl.ds(base, _L)] = acc_x12
                sel_y2[pl.ds(base, _L)] = acc_y22
                sel_x2[pl.ds(base, _L)] = acc_x22

            a1 = (by2 - by1) * (bx2 - bx1)

            def sweep4(i, c):
                slots = []
                for u in range(4):
                    mx, mi, py1, px1, py2, px2 = c[6 * u:6 * u + 6]
                    blk = i * 4 + u
                    sl = pl.ds(blk * _L, _L)
                    y1 = y1v[sl]
                    x1 = x1v[sl]
                    y2 = y2v[sl]
                    x2 = x2v[sl]
                    yA = jnp.maximum(by1, y1)
                    xA = jnp.maximum(bx1, x1)
                    yB = jnp.minimum(by2, y2)
                    xB = jnp.minimum(bx2, x2)
                    inter = (jnp.maximum(yB - yA, 0.0)
                             * jnp.maximum(xB - xA, 0.0))
                    iou = inter / (a1 + a2v[sl] - inter + 1e-8)
                    gidx = jnp.broadcast_to(blk * _L, (_L,)) + iota
                    s = sv[sl]
                    snew = jnp.where((iou > _IOU_THRES) | (gidx == giv),
                                     _NEG, s)
                    sv[sl] = snew
                    take = snew > mx
                    slots.extend(
                        (jnp.where(take, snew, mx),
                         jnp.where(take, gidx, mi),
                         jnp.where(take, y1, py1),
                         jnp.where(take, x1, px1),
                         jnp.where(take, y2, py2),
                         jnp.where(take, x2, px2)))
                return tuple(slots)

            zf = jnp.zeros((_L,), f32)
            init1 = (jnp.full((_L,), -2e10, f32),
                     jnp.zeros((_L,), jnp.int32), zf, zf, zf, zf)
            out4 = lax.fori_loop(0, NB // 4, sweep4, init1 * 4)

            def comb(sa, sb):
                take = ((sb[0] > sa[0])
                        | ((sb[0] == sa[0]) & (sb[1] < sa[1])))
                return tuple(jnp.where(take, b, a)
                             for a, b in zip(sa, sb))

            s01 = comb(out4[0:6], out4[6:12])
            s23 = comb(out4[12:18], out4[18:24])
            mx, mi, py1, px1, py2, px2 = comb(s01, s23)
            nMv, ngiv, (nby1, nbx1, nby2, nbx2) = _argmax_bcast(
                mx, mi, [py1, px1, py2, px2], rotf, roti)

            flvf = jnp.broadcast_to(jnp.where(flush, 1.0, 0.0), (_L,))
            keepf = 1.0 - flvf
            return (nMv, ngiv, nby1, nbx1, nby2, nbx2,
                    acc_s2 * keepf - flvf,
                    acc_y12 * keepf,
                    acc_x12 * keepf,
                    acc_y22 * keepf,
                    acc_x22 * keepf)

        lax.fori_loop(0, _MAX_BOX, step,
                      (Mv0, giv0, by10, bx10, by20, bx20,
                       jnp.full((_L,), -1.0, f32), zf0, zf0, zf0, zf0))

        pltpu.sync_copy(sel_s, sels_hbm.at[pl.ds(r * _KP, _KP)])
        pltpu.sync_copy(sel_y1, sy1_hbm.at[pl.ds(r * _KP, _KP)])
        pltpu.sync_copy(sel_x1, sx1_hbm.at[pl.ds(r * _KP, _KP)])
        pltpu.sync_copy(sel_y2, sy2_hbm.at[pl.ds(r * _KP, _KP)])
        pltpu.sync_copy(sel_x2, sx2_hbm.at[pl.ds(r * _KP, _KP)])

    for j in range(-(-RS // NW)):
        r = wid + NW * j

        @pl.when(r < RS)
        def _():
            do_class(r)


def _nms_sc_pair_body(C, RS, B, BP, H, W,
                      sc_hbm, ro_hbm, dl_hbm,
                      sels_hbm, sy1_hbm, sx1_hbm, sy2_hbm, sx2_hbm,
                      sv, ry1, rx1, ry2, rx2, d0, d1, d2, d3,
                      y1v, x1v, y2v, x2v, a2v,
                      sel_s, sel_y1, sel_x1, sel_y2, sel_x2,
                      rotf, roti, xf, xf2, xi, xi2, shf, shi):
    HB = BP // 4
    NBH = HB // _L
    f32 = jnp.float32
    iota = lax.iota(jnp.int32, _L)
    c = lax.axis_index("c")
    s = lax.axis_index("s")
    h = s - (s // 4) * 4
    r = c * (RS // 2) + s // 4
    n = r // C
    base = h * HB

    pltpu.sync_copy(sc_hbm.at[pl.ds(r * BP + base, HB)], sv)
    pltpu.sync_copy(ro_hbm.at[pl.ds((n * 4 + 0) * BP + base, HB)], ry1)
    pltpu.sync_copy(ro_hbm.at[pl.ds((n * 4 + 1) * BP + base, HB)], rx1)
    pltpu.sync_copy(ro_hbm.at[pl.ds((n * 4 + 2) * BP + base, HB)], ry2)
    pltpu.sync_copy(ro_hbm.at[pl.ds((n * 4 + 3) * BP + base, HB)], rx2)
    pltpu.sync_copy(dl_hbm.at[pl.ds((r * 4 + 0) * BP + base, HB)], d0)
    pltpu.sync_copy(dl_hbm.at[pl.ds((r * 4 + 1) * BP + base, HB)], d1)
    pltpu.sync_copy(dl_hbm.at[pl.ds((r * 4 + 2) * BP + base, HB)], d2)
    pltpu.sync_copy(dl_hbm.at[pl.ds((r * 4 + 3) * BP + base, HB)], d3)

    def exchange(kk, Mv, giv, by1, bx1, by2, bx2):
        buf = kk - (kk // 2) * 2
        xf[pl.ds(0, _L)] = Mv
        xf[pl.ds(_L, _L)] = by1
        xf[pl.ds(2 * _L, _L)] = bx1
        xf[pl.ds(3 * _L, _L)] = by2
        xf[pl.ds(4 * _L, _L)] = bx2
        xi[pl.ds(0, _L)] = giv
        slot = buf * 16 + s
        pltpu.sync_copy(xf, shf.at[pl.ds(slot * 5 * _L, 5 * _L)])
        pltpu.sync_copy(xi, shi.at[pl.ds(slot * _L, _L)])
        plsc.subcore_barrier()
        for dq in (1, 2, 3):
            pslot = buf * 16 + jnp.bitwise_xor(s, dq)
            pltpu.sync_copy(shf.at[pl.ds(pslot * 5 * _L, 5 * _L)], xf2)
            pltpu.sync_copy(shi.at[pl.ds(pslot * _L, _L)], xi2)
            pMv = xf2[pl.ds(0, _L)]
            pby1 = xf2[pl.ds(_L, _L)]
            pbx1 = xf2[pl.ds(2 * _L, _L)]
            pby2 = xf2[pl.ds(3 * _L, _L)]
            pbx2 = xf2[pl.ds(4 * _L, _L)]
            pgiv = xi2[pl.ds(0, _L)]
            take = (pMv > Mv) | ((pMv == Mv) & (pgiv < giv))
            Mv = jnp.where(take, pMv, Mv)
            giv = jnp.where(take, pgiv, giv)
            by1 = jnp.where(take, pby1, by1)
            bx1 = jnp.where(take, pbx1, bx1)
            by2 = jnp.where(take, pby2, by2)
            bx2 = jnp.where(take, pbx2, bx2)
        return (Mv, giv, by1, bx1, by2, bx2)

    def dec(i, carry):
        mx, mi, py1, px1, py2, px2 = carry
        sl = pl.ds(i * _L, _L)
        a = ry1[sl]
        b = rx1[sl]
        c2 = ry2[sl]
        e = rx2[sl]
        w0 = e - b + 1.0
        h0 = c2 - a + 1.0
        x0 = b + w0 / 2.0
        y0 = a + h0 / 2.0
        cx = (d0[sl] / 10.0) * w0 + x0
        cy = (d1[sl] / 10.0) * h0 + y0
        wwv = jnp.exp(d2[sl] / 5.0) * w0
        hhv = jnp.exp(d3[sl] / 5.0) * h0
        xx1 = jnp.clip(cx - 0.5 * wwv, 0.0, W - 1.0)
        yy1 = jnp.clip(cy - 0.5 * hhv, 0.0, H - 1.0)
        xx2 = jnp.clip(cx + 0.5 * wwv, 0.0, W - 1.0)
        yy2 = jnp.clip(cy + 0.5 * hhv, 0.0, H - 1.0)
        y1v[sl] = yy1
        x1v[sl] = xx1
        y2v[sl] = yy2
        x2v[sl] = xx2
        a2v[sl] = (yy2 - yy1) * (xx2 - xx1)
        sc = sv[sl]
        gidx = jnp.broadcast_to(base + i * _L, (_L,)) + iota
        keep = (sc > _SCORE_THRES) & (gidx < B)
        snew = jnp.where(keep, sc, _NEG)
        sv[sl] = snew
        take = snew > mx
        return (jnp.where(take, snew, mx), jnp.where(take, gidx, mi),
                jnp.where(take, yy1, py1), jnp.where(take, xx1, px1),
                jnp.where(take, yy2, py2), jnp.where(take, xx2, px2))

    zf0 = jnp.zeros((_L,), f32)
    mx, mi, py1, px1, py2, px2 = lax.fori_loop(
        0, NBH, dec,
        (jnp.full((_L,), -2e10, f32), jnp.zeros((_L,), jnp.int32),
         zf0, zf0, zf0, zf0))
    Mv0, giv0, (by10, bx10, by20, bx20) = _argmax_bcast(
        mx, mi, [py1, px1, py2, px2], rotf, roti)
    Mv0, giv0, by10, bx10, by20, bx20 = exchange(
        0, Mv0, giv0, by10, bx10, by20, bx20)

    def step(k, carry):
        (Mv, giv, by1, bx1, by2, bx2,
         acc_s, acc_y1, acc_x1, acc_y2, acc_x2) = carry

        validv = Mv > (_NEG * 0.5)
        kmod = k - (k // _L) * _L
        hit = iota == jnp.broadcast_to(kmod, (_L,))
        acc_s2 = jnp.where(hit, jnp.where(validv, Mv, 0.0), acc_s)
        acc_y12 = jnp.where(hit, jnp.where(validv, by1, 0.0), acc_y1)
        acc_x12 = jnp.where(hit, jnp.where(validv, bx1, 0.0), acc_x1)
        acc_y22 = jnp.where(hit, jnp.where(validv, by2, 0.0), acc_y2)
        acc_x22 = jnp.where(hit, jnp.where(validv, bx2, 0.0), acc_x2)
        flush = (kmod == _L - 1) | (k == _MAX_BOX - 1)

        @pl.when(flush)
        def _():
            fbase = k - kmod
            sel_s[pl.ds(fbase, _L)] = acc_s2
            sel_y1[pl.ds(fbase, _L)] = acc_y12
            sel_x1[pl.ds(fbase, _L)] = acc_x12
            sel_y2[pl.ds(fbase, _L)] = acc_y22
            sel_x2[pl.ds(fbase, _L)] = acc_x22

        a1 = (by2 - by1) * (bx2 - bx1)

        def sweep4(i, cc):
            slots = []
            for u in range(4):
                mx, mi, py1, px1, py2, px2 = cc[6 * u:6 * u + 6]
                blk = i * 4 + u
                sl = pl.ds(blk * _L, _L)
                y1 = y1v[sl]
                x1 = x1v[sl]
                y2 = y2v[sl]
                x2 = x2v[sl]
                yA = jnp.maximum(by1, y1)
                xA = jnp.maximum(bx1, x1)
                yB = jnp.minimum(by2, y2)
                xB = jnp.minimum(bx2, x2)
                inter = (jnp.maximum(yB - yA, 0.0)
                         * jnp.maximum(xB - xA, 0.0))
                iou = inter / (a1 + a2v[sl] - inter + 1e-8)
                gidx = jnp.broadcast_to(base + blk * _L, (_L,)) + iota
                sc = sv[sl]
                snew = jnp.where((iou > _IOU_THRES) | (gidx == giv),
                                 _NEG, sc)
                sv[sl] = snew
                take = snew > mx
                slots.extend(
                    (jnp.where(take, snew, mx),
                     jnp.where(take, gidx, mi),
                     jnp.where(take, y1, py1),
                     jnp.where(take, x1, px1),
                     jnp.where(take, y2, py2),
                     jnp.where(take, x2, px2)))
            return tuple(slots)

        zf = jnp.zeros((_L,), f32)
        init1 = (jnp.full((_L,), -2e10, f32),
                 jnp.zeros((_L,), jnp.int32), zf, zf, zf, zf)
        out4 = lax.fori_loop(0, NBH // 4, sweep4, init1 * 4)

        def comb(sa, sb):
            take = ((sb[0] > sa[0])
                    | ((sb[0] == sa[0]) & (sb[1] < sa[1])))
            return tuple(jnp.where(take, b, a) for a, b in zip(sa, sb))

        s01 = comb(out4[0:6], out4[6:12])
        s23 = comb(out4[12:18], out4[18:24])
        mx, mi, py1, px1, py2, px2 = comb(s01, s23)
        nMv, ngiv, (nby1, nbx1, nby2, nbx2) = _argmax_bcast(
            mx, mi, [py1, px1, py2, px2], rotf, roti)
        nMv, ngiv, nby1, nbx1, nby2, nbx2 = exchange(
            k + 1, nMv, ngiv, nby1, nbx1, nby2, nbx2)

        flvf = jnp.broadcast_to(jnp.where(flush, 1.0, 0.0), (_L,))
        keepf = 1.0 - flvf
        return (nMv, ngiv, nby1, nbx1, nby2, nbx2,
                acc_s2 * keepf - flvf,
                acc_y12 * keepf,
                acc_x12 * keepf,
                acc_y22 * keepf,
                acc_x22 * keepf)

    lax.fori_loop(0, _MAX_BOX, step,
                  (Mv0, giv0, by10, bx10, by20, bx20,
                   jnp.full((_L,), -1.0, f32), zf0, zf0, zf0, zf0))

    @pl.when(h == 0)
    def _():
        pltpu.sync_copy(sel_s, sels_hbm.at[pl.ds(r * _KP, _KP)])
        pltpu.sync_copy(sel_y1, sy1_hbm.at[pl.ds(r * _KP, _KP)])
        pltpu.sync_copy(sel_x1, sx1_hbm.at[pl.ds(r * _KP, _KP)])
        pltpu.sync_copy(sel_y2, sy2_hbm.at[pl.ds(r * _KP, _KP)])
        pltpu.sync_copy(sel_x2, sx2_hbm.at[pl.ds(r * _KP, _KP)])


def _nms_tc_body(H, W, B,
                 s_in, ry1, rx1, ry2, rx2, t0, t1, t2, t3,
                 oss, osy1, osx1, osy2, osx2,
                 sref, y1s, x1s, y2s, x2s, a2s):
    RT, BPT = s_in.shape
    f32 = jnp.float32

    w0 = rx2[...] - rx1[...] + 1.0
    h0 = ry2[...] - ry1[...] + 1.0
    x0 = rx1[...] + w0 / 2.0
    y0 = ry1[...] + h0 / 2.0
    cx = (t0[...] / 10.0) * w0 + x0
    cy = (t1[...] / 10.0) * h0 + y0
    ww = jnp.exp(t2[...] / 5.0) * w0
    hh = jnp.exp(t3[...] / 5.0) * h0
    xx1 = jnp.clip(cx - 0.5 * ww, 0.0, W - 1.0)
    yy1 = jnp.clip(cy - 0.5 * hh, 0.0, H - 1.0)
    xx2 = jnp.clip(cx + 0.5 * ww, 0.0, W - 1.0)
    yy2 = jnp.clip(cy + 0.5 * hh, 0.0, H - 1.0)
    y1s[...] = yy1
    x1s[...] = xx1
    y2s[...] = yy2
    x2s[...] = xx2
    a2s[...] = (yy2 - yy1) * (xx2 - xx1)

    lane = jax.lax.broadcasted_iota(jnp.int32, (RT, BPT), 1)
    s = s_in[...]
    sref[...] = jnp.where((lane < B) & (s > _SCORE_THRES), s, _NEG)

    lane128 = jax.lax.broadcasted_iota(jnp.int32, (RT, 128), 1)

    def nms_step(k, carry):
        ss, sy1, sx1, sy2, sx2 = carry
        s = sref[...]
        best = jnp.max(s, axis=1, keepdims=True)
        idx = jnp.min(jnp.where(s == best, lane, BPT), axis=1,
                      keepdims=True)
        eq = lane == idx
        eqf = eq.astype(f32)
        y1 = y1s[...]
        x1 = x1s[...]
        y2 = y2s[...]
        x2 = x2s[...]
        by1 = jnp.sum(y1 * eqf, axis=1, keepdims=True)
        bx1 = jnp.sum(x1 * eqf, axis=1, keepdims=True)
        by2 = jnp.sum(y2 * eqf, axis=1, keepdims=True)
        bx2 = jnp.sum(x2 * eqf, axis=1, keepdims=True)
        yA = jnp.maximum(by1, y1)
        xA = jnp.maximum(bx1, x1)
        yB = jnp.minimum(by2, y2)
        xB = jnp.minimum(bx2, x2)
        inter = jnp.maximum(yB - yA, 0.0) * jnp.maximum(xB - xA, 0.0)
        a1 = (by2 - by1) * (bx2 - bx1)
        iou = inter / (a1 + a2s[...] - inter + 1e-8)
        sref[...] = jnp.where((iou > _IOU_THRES) | eq, _NEG, s)
        valid = best > _NEG * 0.5
        hit = lane128 == k
        ss = jnp.where(hit, jnp.where(valid, best, 0.0), ss)
        sy1 = jnp.where(hit, jnp.where(valid, by1, 0.0), sy1)
        sx1 = jnp.where(hit, jnp.where(valid, bx1, 0.0), sx1)
        sy2 = jnp.where(hit, jnp.where(valid, by2, 0.0), sy2)
        sx2 = jnp.where(hit, jnp.where(valid, bx2, 0.0), sx2)
        return ss, sy1, sx1, sy2, sx2

    init = (jnp.full((RT, 128), -1.0, f32),
            jnp.zeros((RT, 128), f32), jnp.zeros((RT, 128), f32),
            jnp.zeros((RT, 128), f32), jnp.zeros((RT, 128), f32))
    ss, sy1, sx1, sy2, sx2 = jax.lax.fori_loop(0, _MAX_BOX, nms_step, init)
    oss[...] = ss
    osy1[...] = sy1
    osx1[...] = sx1
    osy2[...] = sy2
    osx2[...] = sx2


def _merge_sc_body(C, N,
                   sels_hbm, sy1_hbm, sx1_hbm, sy2_hbm, sx2_hbm,
                   oy1_hbm, ox1_hbm, oy2_hbm, ox2_hbm, osc_hbm, ocl_hbm,
                   ond_hbm,
                   msv, mby1, mbx1, mby2, mbx2,
                   vy1, vx1, vy2, vx2, vsc, vcl, vnd, rotf, roti):
    f32 = jnp.float32
    iota = lax.iota(jnp.int32, _L)
    NBK = C * _KP // _L
    wid = lax.axis_index("s") * 2 + lax.axis_index("c")

    @pl.when(wid < N)
    def _():
        n = wid
        CK = C * _KP
        pltpu.sync_copy(sels_hbm.at[pl.ds(n * CK, CK)], msv)
        pltpu.sync_copy(sy1_hbm.at[pl.ds(n * CK, CK)], mby1)
        pltpu.sync_copy(sx1_hbm.at[pl.ds(n * CK, CK)], mbx1)
        pltpu.sync_copy(sy2_hbm.at[pl.ds(n * CK, CK)], mby2)
        pltpu.sync_copy(sx2_hbm.at[pl.ds(n * CK, CK)], mbx2)

        def mstep(k, carry):
            (lastMv, lastPv, nd,
             ac_s, ac_y1, ac_x1, ac_y2, ac_x2, ac_cl) = carry

            NPB = _KP // _L

            def amax(t, c):
                mx, mp, p1, p2, p3, p4, p5 = c
                sl = pl.ds(t * _L, _L)
                v = msv[sl]
                pos = jnp.broadcast_to(t * _L, (_L,)) + iota
                row = t // NPB
                rowf = jnp.broadcast_to(row, (_L,)).astype(f32)
                elig = (v < lastMv) | ((v == lastMv) & (pos > lastPv))
                take = elig & (v > mx)
                return (jnp.where(take, v, mx), jnp.where(take, pos, mp),
                        jnp.where(take, mby1[sl], p1),
                        jnp.where(take, mbx1[sl], p2),
                        jnp.where(take, mby2[sl], p3),
                        jnp.where(take, mbx2[sl], p4),
                        jnp.where(take, rowf, p5))

            zf = jnp.zeros((_L,), f32)
            mx, mp, p1, p2, p3, p4, p5 = lax.fori_loop(
                0, NBK, amax,
                (jnp.full((_L,), -4.0, f32), jnp.zeros((_L,), jnp.int32),
                 zf, zf, zf, zf, zf))
            Mv, pickv, (by1, bx1, by2, bx2, clsv) = _argmax_bcast(
                mx, mp, [p1, p2, p3, p4, p5], rotf, roti)
            validv = Mv > 0.0
            kmod = k - (k // _L) * _L
            hit = iota == jnp.broadcast_to(kmod, (_L,))
            ac_s2 = jnp.where(hit, Mv, ac_s)
            ac_y12 = jnp.where(hit, jnp.where(validv, by1, 0.0), ac_y1)
            ac_x12 = jnp.where(hit, jnp.where(validv, bx1, 0.0), ac_x1)
            ac_y22 = jnp.where(hit, jnp.where(validv, by2, 0.0), ac_y2)
            ac_x22 = jnp.where(hit, jnp.where(validv, bx2, 0.0), ac_x2)
            ac_cl2 = jnp.where(hit, jnp.where(validv, clsv, 0.0), ac_cl)
            flush = (kmod == _L - 1) | (k == _MAX_BOX - 1)

            @pl.when(flush)
            def _():
                base = k - kmod
                vsc[pl.ds(base, _L)] = ac_s2
                vy1[pl.ds(base, _L)] = ac_y12
                vx1[pl.ds(base, _L)] = ac_x12
                vy2[pl.ds(base, _L)] = ac_y22
                vx2[pl.ds(base, _L)] = ac_x22
                vcl[pl.ds(base, _L)] = ac_cl2

            flvf = jnp.broadcast_to(jnp.where(flush, 1.0, 0.0), (_L,))
            keepf = 1.0 - flvf
            return (Mv, pickv, nd + jnp.where(Mv > 0.0, 1, 0),
                    ac_s2 * keepf,
                    ac_y12 * keepf,
                    ac_x12 * keepf,
                    ac_y22 * keepf,
                    ac_x22 * keepf,
                    ac_cl2 * keepf)

        zfm = jnp.zeros((_L,), f32)
        out_carry = lax.fori_loop(
            0, _MAX_BOX, mstep,
            (jnp.full((_L,), 3.4e38, f32), jnp.full((_L,), -1, jnp.int32),
             jnp.zeros((_L,), jnp.int32), zfm, zfm, zfm, zfm, zfm, zfm))
        nd = out_carry[2]

        vnd[...] = nd
        pltpu.sync_copy(vy1, oy1_hbm.at[pl.ds(n * _KP, _KP)])
        pltpu.sync_copy(vx1, ox1_hbm.at[pl.ds(n * _KP, _KP)])
        pltpu.sync_copy(vy2, oy2_hbm.at[pl.ds(n * _KP, _KP)])
        pltpu.sync_copy(vx2, ox2_hbm.at[pl.ds(n * _KP, _KP)])
        pltpu.sync_copy(vsc, osc_hbm.at[pl.ds(n * _KP, _KP)])
        pltpu.sync_copy(vcl, ocl_hbm.at[pl.ds(n * _KP, _KP)])
        pltpu.sync_copy(vnd, ond_hbm.at[pl.ds(n * _L, _L)])


def kernel(rois, rcnn_conf, rcnn_deltas, input_image):
    N, B = rois.shape[0], rois.shape[1]
    C = rcnn_conf.shape[2] - 1
    H = float(input_image.shape[2])
    W = float(input_image.shape[3])
    R = N * C
    BP = ((B + 16 * _L - 1) // (16 * _L)) * (16 * _L)
    padw = BP - B
    f = jnp.float32

    score_rb = jnp.transpose(rcnn_conf[:, :, :C], (0, 2, 1)).reshape(R, B)
    dd_rb = jnp.transpose(rcnn_deltas.reshape(N, B, C, 4),
                          (0, 2, 3, 1)).reshape(R, 4, B)
    ro_rb = jnp.transpose(rois, (0, 2, 1))

    NW = 32
    RS = min(R, NW // 4)
    RT = R - RS

    score_t = jnp.pad(score_rb, ((0, 0), (0, padw))).reshape(-1)
    ro_t = jnp.pad(ro_rb, ((0, 0), (0, 0), (0, padw))).reshape(-1)
    dl_t = jnp.pad(dd_rb, ((0, 0), (0, 0), (0, padw))).reshape(-1)

    mesh = plsc.VectorSubcoreMesh(core_axis_name="c", subcore_axis_name="s",
                                  num_cores=2, num_subcores=16)

    k1 = pl.kernel(
        functools.partial(_nms_sc_pair_body, C, RS, B, BP, H, W),
        out_type=[jax.ShapeDtypeStruct((RS * _KP,), f) for _ in range(5)],
        mesh=mesh,
        scratch_types=([pltpu.VMEM((BP // 4,), f) for _ in range(14)]
                       + [pltpu.VMEM((_KP,), f) for _ in range(5)]
                       + [pltpu.VMEM((2 * _L,), f),
                          pltpu.VMEM((2 * _L,), jnp.int32)]
                       + [pltpu.VMEM((5 * _L,), f), pltpu.VMEM((5 * _L,), f),
                          pltpu.VMEM((_L,), jnp.int32),
                          pltpu.VMEM((_L,), jnp.int32),
                          pltpu.VMEM_SHARED((2 * 16 * 5 * _L,), f),
                          pltpu.VMEM_SHARED((2 * 16 * _L,), jnp.int32)]),
    )
    sc_out = k1(score_t, ro_t, dl_t)

    BPT = ((B + 127) // 128) * 128
    padt = BPT - B

    def pt(x):
        return jnp.pad(x, ((0, 0), (0, padt)))

    rb_t = jnp.broadcast_to(rois[:, None, :, :], (N, C, B, 4))
    rb_t = rb_t.reshape(R, B, 4)[RS:]
    args_tc = [pt(score_rb[RS:]),
               pt(rb_t[:, :, 0]), pt(rb_t[:, :, 1]),
               pt(rb_t[:, :, 2]), pt(rb_t[:, :, 3]),
               pt(dd_rb[RS:, 0, :]), pt(dd_rb[RS:, 1, :]),
               pt(dd_rb[RS:, 2, :]), pt(dd_rb[RS:, 3, :])]
    tc_out = pl.pallas_call(
        functools.partial(_nms_tc_body, H, W, B),
        out_shape=[jax.ShapeDtypeStruct((RT, 128), f) for _ in range(5)],
        scratch_shapes=[pltpu.VMEM((RT, BPT), f) for _ in range(6)],
    )(*args_tc)

    merged_in = [
        jnp.concatenate([a.reshape(RS, _KP), b[:, :_KP]], axis=0).reshape(-1)
        for a, b in zip(sc_out, tc_out)]
    sels, sy1, sx1, sy2, sx2 = merged_in

    k2 = pl.kernel(
        functools.partial(_merge_sc_body, C, N),
        out_type=([jax.ShapeDtypeStruct((N * _KP,), f) for _ in range(6)]
                  + [jax.ShapeDtypeStruct((N * _L,), jnp.int32)]),
        mesh=mesh,
        scratch_types=([pltpu.VMEM((C * _KP,), f) for _ in range(5)]
                       + [pltpu.VMEM((_KP,), f) for _ in range(6)]
                       + [pltpu.VMEM((_L,), jnp.int32)]
                       + [pltpu.VMEM((2 * _L,), f),
                          pltpu.VMEM((2 * _L,), jnp.int32)]),
    )
    oy1, ox1, oy2, ox2, osc, ocl, ond = k2(sels, sy1, sx1, sy2, sx2)

    oy1 = oy1.reshape(N, _KP)
    ox1 = ox1.reshape(N, _KP)
    oy2 = oy2.reshape(N, _KP)
    ox2 = ox2.reshape(N, _KP)
    osc = osc.reshape(N, _KP)
    ocl = ocl.reshape(N, _KP)
    ond = ond.reshape(N, _L)

    nmsed_boxes = jnp.stack(
        [oy1[:, :_MAX_BOX], ox1[:, :_MAX_BOX],
         oy2[:, :_MAX_BOX], ox2[:, :_MAX_BOX]], axis=-1)
    return (nmsed_boxes, osc[:, :_MAX_BOX], ocl[:, :_MAX_BOX], ond[:, 0],
            rois)
